# Initial kernel scaffold; baseline (speedup 1.0000x reference)
import jax
import jax.numpy as jnp
from jax.experimental import pallas as pl


def kernel(loc, scores, dbox_list):
    raise NotImplementedError("write your pallas kernel here")


# speedup vs baseline: 14.2285x; 14.2285x over previous
import jax
import jax.numpy as jnp
from jax.experimental import pallas as pl
from jax.experimental.pallas import tpu as pltpu

_OVERLAP = 0.45
_TOP_K = 200
_N = 20000
_ROWS = 160
_LANES = 128
_NPAD = _ROWS * _LANES
_IMIN = jnp.iinfo(jnp.int32).min


def _nms_kernel(loc_ref, sco_ref, dbox_ref, box_ref, keep_ref, cnt_ref,
                sbits_ref, x1s, y1s, x2s, y2s, ars,
                cs_ref, ci_ref, c1_ref, c2_ref, c3_ref, c4_ref, ca_ref,
                act_ref):
    f32 = jnp.float32
    l0 = loc_ref[0]
    l1 = loc_ref[1]
    l2 = loc_ref[2]
    l3 = loc_ref[3]
    d0 = dbox_ref[0]
    d1 = dbox_ref[1]
    d2 = dbox_ref[2]
    d3 = dbox_ref[3]
    cx = d0 + (l0 * f32(0.1)) * d2
    cy = d1 + (l1 * f32(0.1)) * d3
    w = d2 * jnp.exp(l2 * f32(0.2))
    h = d3 * jnp.exp(l3 * f32(0.2))
    x1 = cx - w / f32(2.0)
    y1 = cy - h / f32(2.0)
    x2 = x1 + w
    y2 = y1 + h
    box_ref[0] = x1
    box_ref[1] = y1
    box_ref[2] = x2
    box_ref[3] = y2
    x1s[...] = x1
    y1s[...] = y1
    x2s[...] = x2
    y2s[...] = y2
    ars[...] = (x2 - x1) * (y2 - y1)
    sbits_ref[...] = pltpu.bitcast(sco_ref[...], jnp.int32)

    flat = jax.lax.broadcasted_iota(jnp.int32, (_ROWS, _LANES), 0) * _LANES + \
        jax.lax.broadcasted_iota(jnp.int32, (_ROWS, _LANES), 1)
    lane1 = jax.lax.broadcasted_iota(jnp.int32, (1, _LANES), 1)
    cpos = jax.lax.broadcasted_iota(jnp.int32, (8, _LANES), 0) * _LANES + \
        jax.lax.broadcasted_iota(jnp.int32, (8, _LANES), 1)

    cs_ref[...] = jnp.full((8, _LANES), _IMIN, jnp.int32)
    ci_ref[...] = jnp.full((8, _LANES), -1, jnp.int32)
    z8 = jnp.zeros((8, _LANES), f32)
    c1_ref[...] = z8
    c2_ref[...] = z8
    c3_ref[...] = z8
    c4_ref[...] = z8
    ca_ref[...] = z8
    keep_ref[...] = jnp.zeros((8, _LANES), jnp.int32)

    def _pick_row(ref, r, c):
        row = ref[pl.ds(r, 1), :]
        return jnp.sum(jnp.where(lane1 == c, row, f32(0.0)))

    def ext_body(j, carry):
        sb = sbits_ref[...]
        m = jnp.max(sb)
        i = jnp.max(jnp.where(sb == m, flat, -1))
        sbits_ref[...] = jnp.where(flat == i, _IMIN, sb)
        r = jax.lax.shift_right_logical(i, 7)
        c = jax.lax.bitwise_and(i, 127)
        sel = cpos == j
        cs_ref[...] = jnp.where(sel, m, cs_ref[...])
        ci_ref[...] = jnp.where(sel, i, ci_ref[...])
        c1_ref[...] = jnp.where(sel, _pick_row(x1s, r, c), c1_ref[...])
        c2_ref[...] = jnp.where(sel, _pick_row(y1s, r, c), c2_ref[...])
        c3_ref[...] = jnp.where(sel, _pick_row(x2s, r, c), c3_ref[...])
        c4_ref[...] = jnp.where(sel, _pick_row(y2s, r, c), c4_ref[...])
        ca_ref[...] = jnp.where(sel, _pick_row(ars, r, c), ca_ref[...])
        return carry

    jax.lax.fori_loop(0, _TOP_K, ext_body, jnp.int32(0))

    act_ref[...] = (cpos < _TOP_K).astype(jnp.int32)

    def cond(st):
        return st[1]

    def body(st):
        count, _ = st
        active = act_ref[...] != 0
        cs = cs_ref[...]
        ci = ci_ref[...]
        c1 = c1_ref[...]
        c2 = c2_ref[...]
        c3 = c3_ref[...]
        c4 = c4_ref[...]
        ca = ca_ref[...]
        m = jnp.max(jnp.where(active, cs, _IMIN))
        i = jnp.max(jnp.where(active & (cs == m), ci, -1))
        pm = ci == i
        bx1 = jnp.max(jnp.where(pm, c1, -jnp.inf))
        by1 = jnp.max(jnp.where(pm, c2, -jnp.inf))
        bx2 = jnp.max(jnp.where(pm, c3, -jnp.inf))
        by2 = jnp.max(jnp.where(pm, c4, -jnp.inf))
        bar = jnp.max(jnp.where(pm, ca, -jnp.inf))
        tw = jnp.maximum(jnp.minimum(c3, bx2) - jnp.maximum(c1, bx1), f32(0.0))
        th = jnp.maximum(jnp.minimum(c4, by2) - jnp.maximum(c2, by1), f32(0.0))
        inter = tw * th
        iou = inter / (ca - inter + bar)
        active = active & (iou <= f32(_OVERLAP))
        act_ref[...] = active.astype(jnp.int32)
        keep_ref[...] = jnp.where(cpos == count, i, keep_ref[...])
        return count + 1, jnp.any(active)

    count, _ = jax.lax.while_loop(cond, body, (jnp.int32(0), True))
    cnt_ref[0, 0] = count


def kernel(loc, scores, dbox_list):
    f32 = jnp.float32
    locp = jnp.zeros((4, _NPAD), f32).at[:, :_N].set(loc.T).reshape(
        4, _ROWS, _LANES)
    dbxp = jnp.zeros((4, _NPAD), f32).at[:, :_N].set(dbox_list.T).reshape(
        4, _ROWS, _LANES)
    scop = jnp.full((_NPAD,), -jnp.inf, f32).at[:_N].set(scores).reshape(
        _ROWS, _LANES)
    boxes4, keep8, cnt = pl.pallas_call(
        _nms_kernel,
        out_shape=(
            jax.ShapeDtypeStruct((4, _ROWS, _LANES), f32),
            jax.ShapeDtypeStruct((8, _LANES), jnp.int32),
            jax.ShapeDtypeStruct((1, 1), jnp.int32),
        ),
        out_specs=(
            pl.BlockSpec(),
            pl.BlockSpec(),
            pl.BlockSpec(memory_space=pltpu.SMEM),
        ),
        scratch_shapes=[
            pltpu.VMEM((_ROWS, _LANES), jnp.int32),
            pltpu.VMEM((_ROWS, _LANES), f32),
            pltpu.VMEM((_ROWS, _LANES), f32),
            pltpu.VMEM((_ROWS, _LANES), f32),
            pltpu.VMEM((_ROWS, _LANES), f32),
            pltpu.VMEM((_ROWS, _LANES), f32),
            pltpu.VMEM((8, _LANES), jnp.int32),
            pltpu.VMEM((8, _LANES), jnp.int32),
            pltpu.VMEM((8, _LANES), f32),
            pltpu.VMEM((8, _LANES), f32),
            pltpu.VMEM((8, _LANES), f32),
            pltpu.VMEM((8, _LANES), f32),
            pltpu.VMEM((8, _LANES), f32),
            pltpu.VMEM((8, _LANES), jnp.int32),
        ],
    )(locp, scop, dbxp)
    boxes = boxes4.reshape(4, _NPAD)[:, :_N].T
    keep = jnp.concatenate(
        [keep8.reshape(-1), jnp.zeros((_N - 8 * _LANES,), jnp.int32)])
    return boxes, keep, cnt[0, 0]


# speedup vs baseline: 41.2871x; 2.9017x over previous
import functools

import jax
import jax.numpy as jnp
from jax import lax
from jax.experimental import pallas as pl
from jax.experimental.pallas import tpu as pltpu
from jax.experimental.pallas import tpu_sc as plsc

_OVERLAP = 0.45
_TOP_K = 200
_N = 20000
_ROWS = 160
_LANES = 128
_NPAD = _ROWS * _LANES
_IMIN = jnp.iinfo(jnp.int32).min
_NW = 16
_SH = _NPAD // _NW
_SHV = _SH // 16
_PACK = 2048
_OSB, _OID, _OX1, _OY1, _OX2, _OY2, _OAR = 16, 272, 528, 784, 1040, 1296, 1552
_KPAD = 208
_KV = _KPAD // 16


def _tc_kernel(loc_ref, sco_ref, dbox_ref, box_ref, sbit_ref, tj_ref):
    f32 = jnp.float32
    l0 = loc_ref[0]
    l1 = loc_ref[1]
    l2 = loc_ref[2]
    l3 = loc_ref[3]
    d0 = dbox_ref[0]
    d1 = dbox_ref[1]
    d2 = dbox_ref[2]
    d3 = dbox_ref[3]
    cx = d0 + (l0 * f32(0.1)) * d2
    cy = d1 + (l1 * f32(0.1)) * d3
    w = d2 * jnp.exp(l2 * f32(0.2))
    h = d3 * jnp.exp(l3 * f32(0.2))
    x1 = cx - w / f32(2.0)
    y1 = cy - h / f32(2.0)
    box_ref[0] = x1
    box_ref[1] = y1
    box_ref[2] = x1 + w
    box_ref[3] = y1 + h

    bits = pltpu.bitcast(sco_ref[...], jnp.int32)
    sbit_ref[...] = bits
    flat = lax.broadcasted_iota(jnp.int32, (_ROWS, _LANES), 0) * _LANES + \
        lax.broadcasted_iota(jnp.int32, (_ROWS, _LANES), 1)

    def bs1(_, c):
        lo, hi = c
        mid = lo + ((hi - lo + jnp.int32(1)) >> 1)
        cnt = jnp.sum((bits >= mid).astype(jnp.int32))
        ok = cnt >= _TOP_K
        return jnp.where(ok, mid, lo), jnp.where(ok, hi, mid - 1)

    tb, _ = lax.fori_loop(0, 31, bs1, (jnp.int32(0), jnp.int32(0x3F7FFFFF)))
    k2 = _TOP_K - jnp.sum((bits > tb).astype(jnp.int32))

    def bs2(_, c):
        lo, hi = c
        mid = lo + ((hi - lo + jnp.int32(1)) >> 1)
        cnt = jnp.sum(((bits == tb) & (flat >= mid)).astype(jnp.int32))
        ok = cnt >= k2
        return jnp.where(ok, mid, lo), jnp.where(ok, hi, mid - 1)

    jj, _ = lax.fori_loop(0, 16, bs2, (jnp.int32(0), jnp.int32(_NPAD - 1)))
    tj_ref[0, 0] = tb
    tj_ref[0, 1] = jj


def _sc_kernel(sb_h, x1_h, y1_h, x2_h, y2_h, tj_h, keep_h, cnt_h,
               sb_v, x1_v, y1_v, x2_v, y2_v, tj_v,
               pack_v, cntv, keep_v, spbig, big_v,
               ksb, kid, kx1, ky1, kx2, ky2, kar, act_v):
    f32 = jnp.float32
    i32 = jnp.int32
    wid = lax.axis_index("s")
    base = wid * _SH

    pltpu.sync_copy(sb_h.at[pl.ds(base, _SH)], sb_v)
    pltpu.sync_copy(x1_h.at[pl.ds(base, _SH)], x1_v)
    pltpu.sync_copy(y1_h.at[pl.ds(base, _SH)], y1_v)
    pltpu.sync_copy(x2_h.at[pl.ds(base, _SH)], x2_v)
    pltpu.sync_copy(y2_h.at[pl.ds(base, _SH)], y2_v)
    pltpu.sync_copy(tj_h, tj_v)
    tjv = tj_v[pl.ds(0, 16)]
    tb = tjv[0]
    jj = tjv[1]

    z16 = jnp.zeros((16,), i32)
    for v in range(_SHV + 1):
        keep_v[pl.ds(v * 16, 16)] = z16

    @pl.when(wid != 0)
    def _zero_keep():
        pltpu.sync_copy(keep_v.at[pl.ds(0, _SH)],
                        keep_h.at[pl.ds(base, _SH)])

    cnt = i32(0)
    for v in range(_SHV):
        sl = pl.ds(v * 16, 16)
        bits = sb_v[sl]
        gi = lax.iota(i32, 16) + (base + v * 16)
        selm = (bits > tb) | ((bits == tb) & (gi >= jj))
        x1 = x1_v[sl]
        y1 = y1_v[sl]
        x2 = x2_v[sl]
        y2 = y2_v[sl]
        pref = plsc.cumsum(selm.astype(i32))
        tgt = jnp.where(selm, cnt + pref - 1, 255)
        plsc.store_scatter(pack_v, [tgt + _OSB], bits)
        plsc.store_scatter(pack_v, [tgt + _OID], gi)
        plsc.store_scatter(pack_v, [tgt + _OX1], plsc.bitcast(x1, i32))
        plsc.store_scatter(pack_v, [tgt + _OY1], plsc.bitcast(y1, i32))
        plsc.store_scatter(pack_v, [tgt + _OX2], plsc.bitcast(x2, i32))
        plsc.store_scatter(pack_v, [tgt + _OY2], plsc.bitcast(y2, i32))
        plsc.store_scatter(pack_v, [tgt + _OAR],
                           plsc.bitcast((x2 - x1) * (y2 - y1), i32))
        cnt = cnt + pref[15]

    lane = lax.iota(i32, 16)
    pack_v[pl.ds(0, 16)] = jnp.where(lane == 0, cnt, 0)
    pltpu.sync_copy(pack_v, spbig.at[wid])
    plsc.subcore_barrier()

    @pl.when(wid == 0)
    def _phase_b():
        pltpu.sync_copy(spbig, big_v)
        cvec = plsc.load_gather(
            big_v, [lax.iota(i32, 16), jnp.zeros((16,), i32)])
        offs = []
        run = i32(0)
        for w in range(_NW):
            offs.append(run)
            run = run + cvec[w]

        for k in range(_KV):
            qv = lax.iota(i32, 16) + (k * 16)
            wq = jnp.zeros((16,), i32)
            bq = jnp.zeros((16,), i32)
            for w in range(1, _NW):
                m = qv >= offs[w]
                wq = wq + m.astype(i32)
                bq = jnp.where(m, offs[w], bq)
            lq = jnp.minimum(jnp.maximum(qv - bq, 0), 255)
            valid = qv < _TOP_K
            s_k = plsc.load_gather(big_v, [wq, lq + _OSB])
            i_k = plsc.load_gather(big_v, [wq, lq + _OID])
            sl = pl.ds(k * 16, 16)
            ksb[sl] = jnp.where(valid, s_k, _IMIN)
            kid[sl] = jnp.where(valid, i_k, -1)
            kx1[sl] = plsc.bitcast(plsc.load_gather(big_v, [wq, lq + _OX1]), f32)
            ky1[sl] = plsc.bitcast(plsc.load_gather(big_v, [wq, lq + _OY1]), f32)
            kx2[sl] = plsc.bitcast(plsc.load_gather(big_v, [wq, lq + _OX2]), f32)
            ky2[sl] = plsc.bitcast(plsc.load_gather(big_v, [wq, lq + _OY2]), f32)
            kar[sl] = plsc.bitcast(plsc.load_gather(big_v, [wq, lq + _OAR]), f32)
            act_v[sl] = valid.astype(i32)

        def cond(st):
            return st[1]

        def body(st):
            count, _ = st
            accs = jnp.full((16,), _IMIN, i32)
            for k in range(_KV):
                sl = pl.ds(k * 16, 16)
                a = act_v[sl] != 0
                accs = jnp.maximum(accs, jnp.where(a, ksb[sl], _IMIN))
            mm = jnp.max(accs)
            acci = jnp.full((16,), -1, i32)
            for k in range(_KV):
                sl = pl.ds(k * 16, 16)
                a = act_v[sl] != 0
                hit = a & (ksb[sl] == mm)
                acci = jnp.maximum(acci, jnp.where(hit, kid[sl], -1))
            ii = jnp.max(acci)
            accp = jnp.full((16,), -1, i32)
            for k in range(_KV):
                sl = pl.ds(k * 16, 16)
                qv = lax.iota(i32, 16) + (k * 16)
                accp = jnp.maximum(
                    accp, jnp.where(kid[sl] == ii, qv, -1))
            p = jnp.max(accp)
            pv = jnp.zeros((16,), i32) + p
            bx1 = plsc.load_gather(kx1, [pv])[0]
            by1 = plsc.load_gather(ky1, [pv])[0]
            bx2 = plsc.load_gather(kx2, [pv])[0]
            by2 = plsc.load_gather(ky2, [pv])[0]
            bar = plsc.load_gather(kar, [pv])[0]
            anyv = jnp.zeros((16,), i32)
            for k in range(_KV):
                sl = pl.ds(k * 16, 16)
                tw = jnp.maximum(
                    jnp.minimum(kx2[sl], bx2) - jnp.maximum(kx1[sl], bx1),
                    f32(0.0))
                th = jnp.maximum(
                    jnp.minimum(ky2[sl], by2) - jnp.maximum(ky1[sl], by1),
                    f32(0.0))
                inter = tw * th
                iou = inter / (kar[sl] - inter + bar)
                nact = jnp.where((act_v[sl] != 0) & (iou <= f32(_OVERLAP)),
                                 i32(1), i32(0))
                act_v[sl] = nact
                anyv = anyv + nact
            cv = jnp.where(lax.iota(i32, 16) == 0,
                           count, _SH)
            iv = jnp.zeros((16,), i32) + ii
            plsc.store_scatter(keep_v, [cv], iv)
            return count + 1, jnp.max(anyv) > 0

        count, _ = lax.while_loop(cond, body, (i32(0), True))
        cntv[pl.ds(0, 16)] = jnp.where(lax.iota(i32, 16) == 0, count, 0)
        pltpu.sync_copy(cntv, cnt_h)
        pltpu.sync_copy(keep_v.at[pl.ds(0, _SH)],
                        keep_h.at[pl.ds(0, _SH)])


_SC_SCRATCH = [
        pltpu.VMEM((_SH,), jnp.int32),
        pltpu.VMEM((_SH,), jnp.float32),
        pltpu.VMEM((_SH,), jnp.float32),
        pltpu.VMEM((_SH,), jnp.float32),
        pltpu.VMEM((_SH,), jnp.float32),
        pltpu.VMEM((16,), jnp.int32),
        pltpu.VMEM((_PACK,), jnp.int32),
        pltpu.VMEM((16,), jnp.int32),
        pltpu.VMEM((_SH + 16,), jnp.int32),
        pltpu.VMEM_SHARED((_NW, _PACK), jnp.int32),
        pltpu.VMEM((_NW, _PACK), jnp.int32),
        pltpu.VMEM((_KPAD,), jnp.int32),
        pltpu.VMEM((_KPAD,), jnp.int32),
        pltpu.VMEM((_KPAD,), jnp.float32),
        pltpu.VMEM((_KPAD,), jnp.float32),
        pltpu.VMEM((_KPAD,), jnp.float32),
        pltpu.VMEM((_KPAD,), jnp.float32),
        pltpu.VMEM((_KPAD,), jnp.float32),
        pltpu.VMEM((_KPAD,), jnp.int32),
]


@functools.lru_cache(maxsize=1)
def _make_sc_call():
  return functools.partial(
    pl.kernel,
    out_type=(
        jax.ShapeDtypeStruct((_NPAD,), jnp.int32),
        jax.ShapeDtypeStruct((16,), jnp.int32),
    ),
    mesh=plsc.VectorSubcoreMesh(core_axis_name="c", subcore_axis_name="s",
                                num_cores=1),
    compiler_params=pltpu.CompilerParams(needs_layout_passes=False),
    scratch_types=_SC_SCRATCH,
)(_sc_kernel)


def kernel(loc, scores, dbox_list):
    f32 = jnp.float32
    locp = jnp.zeros((4, _NPAD), f32).at[:, :_N].set(loc.T).reshape(
        4, _ROWS, _LANES)
    dbxp = jnp.zeros((4, _NPAD), f32).at[:, :_N].set(dbox_list.T).reshape(
        4, _ROWS, _LANES)
    scop = jnp.full((_NPAD,), -jnp.inf, f32).at[:_N].set(scores).reshape(
        _ROWS, _LANES)
    boxes4, sbit, tj = pl.pallas_call(
        _tc_kernel,
        out_shape=(
            jax.ShapeDtypeStruct((4, _ROWS, _LANES), f32),
            jax.ShapeDtypeStruct((_ROWS, _LANES), jnp.int32),
            jax.ShapeDtypeStruct((1, 2), jnp.int32),
        ),
        out_specs=(
            pl.BlockSpec(),
            pl.BlockSpec(),
            pl.BlockSpec(memory_space=pltpu.SMEM),
        ),
    )(locp, scop, dbxp)
    b4 = boxes4.reshape(4, _NPAD)
    tj16 = jnp.zeros((16,), jnp.int32).at[0].set(tj[0, 0]).at[1].set(tj[0, 1])
    keep_p, cnt16 = _make_sc_call()(
        sbit.reshape(_NPAD), b4[0], b4[1], b4[2], b4[3], tj16)
    boxes = b4[:, :_N].T
    return boxes, keep_p[:_N], cnt16[0]


# speedup vs baseline: 46.4035x; 1.1239x over previous
import functools

import jax
import jax.numpy as jnp
from jax import lax
from jax.experimental import pallas as pl
from jax.experimental.pallas import tpu as pltpu
from jax.experimental.pallas import tpu_sc as plsc

_OVERLAP = 0.45
_TOP_K = 200
_N = 20000
_ROWS = 160
_LANES = 128
_NPAD = _ROWS * _LANES
_IMIN = jnp.iinfo(jnp.int32).min
_NW = 16
_SH = _NPAD // _NW
_SHV = _SH // 16
_PACK = 2048
_OSB, _OID, _OX1, _OY1, _OX2, _OY2, _OAR = 16, 272, 528, 784, 1040, 1296, 1552
_KPAD = 208
_KV = _KPAD // 16


def _tc_kernel(loc_ref, sco_ref, dbox_ref, box_ref, sbit_ref, tj_ref):
    f32 = jnp.float32
    l0 = loc_ref[0]
    l1 = loc_ref[1]
    l2 = loc_ref[2]
    l3 = loc_ref[3]
    d0 = dbox_ref[0]
    d1 = dbox_ref[1]
    d2 = dbox_ref[2]
    d3 = dbox_ref[3]
    cx = d0 + (l0 * f32(0.1)) * d2
    cy = d1 + (l1 * f32(0.1)) * d3
    w = d2 * jnp.exp(l2 * f32(0.2))
    h = d3 * jnp.exp(l3 * f32(0.2))
    x1 = cx - w / f32(2.0)
    y1 = cy - h / f32(2.0)
    box_ref[0] = x1
    box_ref[1] = y1
    box_ref[2] = x1 + w
    box_ref[3] = y1 + h

    bits = pltpu.bitcast(sco_ref[...], jnp.int32)
    sbit_ref[...] = bits
    flat = lax.broadcasted_iota(jnp.int32, (_ROWS, _LANES), 0) * _LANES + \
        lax.broadcasted_iota(jnp.int32, (_ROWS, _LANES), 1)

    def bs1(_, c):
        lo, hi = c
        mid = lo + ((hi - lo + jnp.int32(1)) >> 1)
        cnt = jnp.sum((bits >= mid).astype(jnp.int32))
        ok = cnt >= _TOP_K
        return jnp.where(ok, mid, lo), jnp.where(ok, hi, mid - 1)

    tb, _ = lax.fori_loop(0, 31, bs1, (jnp.int32(0), jnp.int32(0x3F7FFFFF)))
    k2 = _TOP_K - jnp.sum((bits > tb).astype(jnp.int32))

    def bs2(_, c):
        lo, hi = c
        mid = lo + ((hi - lo + jnp.int32(1)) >> 1)
        cnt = jnp.sum(((bits == tb) & (flat >= mid)).astype(jnp.int32))
        ok = cnt >= k2
        return jnp.where(ok, mid, lo), jnp.where(ok, hi, mid - 1)

    jj, _ = lax.fori_loop(0, 16, bs2, (jnp.int32(0), jnp.int32(_NPAD - 1)))
    tj_ref[0, 0] = tb
    tj_ref[0, 1] = jj


def _sc_kernel(sb_h, x1_h, y1_h, x2_h, y2_h, tj_h, keep_h, cnt_h,
               sb_v, x1_v, y1_v, x2_v, y2_v, tj_v,
               pack_v, cntv, keep_v, spbig, big_v,
               ksb, kid, kx1, ky1, kx2, ky2, kar):
    f32 = jnp.float32
    i32 = jnp.int32
    wid = lax.axis_index("s")
    base = wid * _SH

    pltpu.sync_copy(sb_h.at[pl.ds(base, _SH)], sb_v)
    pltpu.sync_copy(x1_h.at[pl.ds(base, _SH)], x1_v)
    pltpu.sync_copy(y1_h.at[pl.ds(base, _SH)], y1_v)
    pltpu.sync_copy(x2_h.at[pl.ds(base, _SH)], x2_v)
    pltpu.sync_copy(y2_h.at[pl.ds(base, _SH)], y2_v)
    pltpu.sync_copy(tj_h, tj_v)
    tjv = tj_v[pl.ds(0, 16)]
    tb = tjv[0]
    jj = tjv[1]

    z16 = jnp.zeros((16,), i32)
    for v in range(_SHV + 1):
        keep_v[pl.ds(v * 16, 16)] = z16

    @pl.when(wid != 0)
    def _zero_keep():
        pltpu.sync_copy(keep_v.at[pl.ds(0, _SH)],
                        keep_h.at[pl.ds(base, _SH)])

    cnt = i32(0)
    for v in range(_SHV):
        sl = pl.ds(v * 16, 16)
        bits = sb_v[sl]
        gi = lax.iota(i32, 16) + (base + v * 16)
        selm = (bits > tb) | ((bits == tb) & (gi >= jj))
        x1 = x1_v[sl]
        y1 = y1_v[sl]
        x2 = x2_v[sl]
        y2 = y2_v[sl]
        pref = plsc.cumsum(selm.astype(i32))
        tgt = jnp.where(selm, cnt + pref - 1, 255)
        plsc.store_scatter(pack_v, [tgt + _OSB], bits)
        plsc.store_scatter(pack_v, [tgt + _OID], gi)
        plsc.store_scatter(pack_v, [tgt + _OX1], plsc.bitcast(x1, i32))
        plsc.store_scatter(pack_v, [tgt + _OY1], plsc.bitcast(y1, i32))
        plsc.store_scatter(pack_v, [tgt + _OX2], plsc.bitcast(x2, i32))
        plsc.store_scatter(pack_v, [tgt + _OY2], plsc.bitcast(y2, i32))
        plsc.store_scatter(pack_v, [tgt + _OAR],
                           plsc.bitcast((x2 - x1) * (y2 - y1), i32))
        cnt = cnt + pref[15]

    lane = lax.iota(i32, 16)
    pack_v[pl.ds(0, 16)] = jnp.where(lane == 0, cnt, 0)
    pltpu.sync_copy(pack_v, spbig.at[wid])
    plsc.subcore_barrier()

    @pl.when(wid == 0)
    def _phase_b():
        pltpu.sync_copy(spbig, big_v)
        cvec = plsc.load_gather(
            big_v, [lax.iota(i32, 16), jnp.zeros((16,), i32)])
        offs = []
        run = i32(0)
        for w in range(_NW):
            offs.append(run)
            run = run + cvec[w]

        for k in range(_KV):
            qv = lax.iota(i32, 16) + (k * 16)
            wq = jnp.zeros((16,), i32)
            bq = jnp.zeros((16,), i32)
            for w in range(1, _NW):
                m = qv >= offs[w]
                wq = wq + m.astype(i32)
                bq = jnp.where(m, offs[w], bq)
            lq = jnp.minimum(jnp.maximum(qv - bq, 0), 255)
            valid = qv < _TOP_K
            s_k = plsc.load_gather(big_v, [wq, lq + _OSB])
            i_k = plsc.load_gather(big_v, [wq, lq + _OID])
            sl = pl.ds(k * 16, 16)
            ksb[sl] = jnp.where(valid, s_k, _IMIN)
            kid[sl] = jnp.where(valid, i_k, -1)
            kx1[sl] = plsc.bitcast(plsc.load_gather(big_v, [wq, lq + _OX1]), f32)
            ky1[sl] = plsc.bitcast(plsc.load_gather(big_v, [wq, lq + _OY1]), f32)
            kx2[sl] = plsc.bitcast(plsc.load_gather(big_v, [wq, lq + _OX2]), f32)
            ky2[sl] = plsc.bitcast(plsc.load_gather(big_v, [wq, lq + _OY2]), f32)
            kar[sl] = plsc.bitcast(plsc.load_gather(big_v, [wq, lq + _OAR]), f32)

        def cond(st):
            return st[1]

        def body(st):
            count, _, mm = st
            accp = jnp.full((16,), -1, i32)
            for k in range(_KV):
                sl = pl.ds(k * 16, 16)
                qv = lax.iota(i32, 16) + (k * 16)
                accp = jnp.maximum(accp, jnp.where(ksb[sl] == mm, qv, -1))
            p = jnp.max(accp)
            pv = jnp.zeros((16,), i32) + p
            ii = plsc.load_gather(kid, [pv])[0]
            bx1 = plsc.load_gather(kx1, [pv])[0]
            by1 = plsc.load_gather(ky1, [pv])[0]
            bx2 = plsc.load_gather(kx2, [pv])[0]
            by2 = plsc.load_gather(ky2, [pv])[0]
            bar = plsc.load_gather(kar, [pv])[0]
            acc = jnp.full((16,), _IMIN, i32)
            for k in range(_KV):
                sl = pl.ds(k * 16, 16)
                tw = jnp.maximum(
                    jnp.minimum(kx2[sl], bx2) - jnp.maximum(kx1[sl], bx1),
                    f32(0.0))
                th = jnp.maximum(
                    jnp.minimum(ky2[sl], by2) - jnp.maximum(ky1[sl], by1),
                    f32(0.0))
                inter = tw * th
                iou = inter / (kar[sl] - inter + bar)
                nk = jnp.where(iou <= f32(_OVERLAP), ksb[sl], _IMIN)
                ksb[sl] = nk
                acc = jnp.maximum(acc, nk)
            mm2 = jnp.max(acc)
            cv = jnp.where(lax.iota(i32, 16) == 0, count, _SH)
            iv = jnp.zeros((16,), i32) + ii
            plsc.store_scatter(keep_v, [cv], iv)
            return count + 1, mm2 > _IMIN, mm2

        acc0 = jnp.full((16,), _IMIN, i32)
        for k in range(_KV):
            acc0 = jnp.maximum(acc0, ksb[pl.ds(k * 16, 16)])
        mm0 = jnp.max(acc0)
        count, _, _ = lax.while_loop(cond, body, (i32(0), True, mm0))
        cntv[pl.ds(0, 16)] = jnp.where(lax.iota(i32, 16) == 0, count, 0)
        pltpu.sync_copy(cntv, cnt_h)
        pltpu.sync_copy(keep_v.at[pl.ds(0, _SH)],
                        keep_h.at[pl.ds(0, _SH)])


_SC_SCRATCH = [
        pltpu.VMEM((_SH,), jnp.int32),
        pltpu.VMEM((_SH,), jnp.float32),
        pltpu.VMEM((_SH,), jnp.float32),
        pltpu.VMEM((_SH,), jnp.float32),
        pltpu.VMEM((_SH,), jnp.float32),
        pltpu.VMEM((16,), jnp.int32),
        pltpu.VMEM((_PACK,), jnp.int32),
        pltpu.VMEM((16,), jnp.int32),
        pltpu.VMEM((_SH + 16,), jnp.int32),
        pltpu.VMEM_SHARED((_NW, _PACK), jnp.int32),
        pltpu.VMEM((_NW, _PACK), jnp.int32),
        pltpu.VMEM((_KPAD,), jnp.int32),
        pltpu.VMEM((_KPAD,), jnp.int32),
        pltpu.VMEM((_KPAD,), jnp.float32),
        pltpu.VMEM((_KPAD,), jnp.float32),
        pltpu.VMEM((_KPAD,), jnp.float32),
        pltpu.VMEM((_KPAD,), jnp.float32),
        pltpu.VMEM((_KPAD,), jnp.float32),
]


@functools.lru_cache(maxsize=1)
def _make_sc_call():
  return functools.partial(
    pl.kernel,
    out_type=(
        jax.ShapeDtypeStruct((_NPAD,), jnp.int32),
        jax.ShapeDtypeStruct((16,), jnp.int32),
    ),
    mesh=plsc.VectorSubcoreMesh(core_axis_name="c", subcore_axis_name="s",
                                num_cores=1),
    compiler_params=pltpu.CompilerParams(needs_layout_passes=False),
    scratch_types=_SC_SCRATCH,
)(_sc_kernel)


def kernel(loc, scores, dbox_list):
    f32 = jnp.float32
    locp = jnp.zeros((4, _NPAD), f32).at[:, :_N].set(loc.T).reshape(
        4, _ROWS, _LANES)
    dbxp = jnp.zeros((4, _NPAD), f32).at[:, :_N].set(dbox_list.T).reshape(
        4, _ROWS, _LANES)
    scop = jnp.full((_NPAD,), -jnp.inf, f32).at[:_N].set(scores).reshape(
        _ROWS, _LANES)
    boxes4, sbit, tj = pl.pallas_call(
        _tc_kernel,
        out_shape=(
            jax.ShapeDtypeStruct((4, _ROWS, _LANES), f32),
            jax.ShapeDtypeStruct((_ROWS, _LANES), jnp.int32),
            jax.ShapeDtypeStruct((1, 2), jnp.int32),
        ),
        out_specs=(
            pl.BlockSpec(),
            pl.BlockSpec(),
            pl.BlockSpec(memory_space=pltpu.SMEM),
        ),
    )(locp, scop, dbxp)
    b4 = boxes4.reshape(4, _NPAD)
    tj16 = jnp.zeros((16,), jnp.int32).at[0].set(tj[0, 0]).at[1].set(tj[0, 1])
    keep_p, cnt16 = _make_sc_call()(
        sbit.reshape(_NPAD), b4[0], b4[1], b4[2], b4[3], tj16)
    boxes = b4[:, :_N].T
    return boxes, keep_p[:_N], cnt16[0]
